# baseline (speedup 1.0000x reference)
import functools

import jax
import jax.numpy as jnp
from jax import lax
from jax.experimental import pallas as pl
from jax.experimental.pallas import tpu as pltpu
from jax.experimental.pallas import tpu_sc as plsc

VOCAB = 1000000
EMBED_DIM = 32
BATCH = 4096
HIST = 50

_NC = 2
_NS = 16
_NW = _NC * _NS

_B = BATCH * HIST
_PER_W = _B // _NW
_CH = 128
_NCH = _PER_W // _CH
_NBUF = 2


def _gather_body(x_hbm, table_hbm, out_hbm, idx_v, rows_v, g_sems, o_sems):
    wid = lax.axis_index("s") * _NC + lax.axis_index("c")
    base = wid * _PER_W

    pltpu.sync_copy(x_hbm.at[wid], idx_v)

    def start_gather(j, b):
        pltpu.async_copy(table_hbm.at[idx_v.at[j]], rows_v.at[b], g_sems.at[b])

    def wait_gather(j, b):
        pltpu.make_async_copy(
            table_hbm.at[idx_v.at[j]], rows_v.at[b], g_sems.at[b]
        ).wait()

    def start_out(j, b):
        pltpu.async_copy(
            rows_v.at[b], out_hbm.at[pl.ds(base + j * _CH, _CH)], o_sems.at[b]
        )

    def wait_out(j, b):
        pltpu.make_async_copy(
            rows_v.at[b], out_hbm.at[pl.ds(base + j * _CH, _CH)], o_sems.at[b]
        ).wait()

    for b in range(_NBUF):
        start_gather(b, b)

    def step(g, _):
        for b in range(_NBUF):
            j = g * _NBUF + b
            wait_gather(j, b)
            start_out(j, b)
            nxt = j + _NBUF

            @pl.when(nxt < _NCH)
            def _():
                wait_out(nxt - _NBUF, b)
                start_gather(nxt, b)

        return _

    lax.fori_loop(0, _NCH // _NBUF, step, None)

    for b in range(_NBUF):
        j = _NCH - _NBUF + b
        wait_out(j, b)


@jax.jit
def _sc_gather(x3, table):
    kern = pl.kernel(
        _gather_body,
        out_type=jax.ShapeDtypeStruct((_B, EMBED_DIM), jnp.float32),
        mesh=plsc.VectorSubcoreMesh(core_axis_name="c", subcore_axis_name="s"),
        scratch_types=[
            pltpu.VMEM((_NCH, _CH), jnp.int32),
            pltpu.VMEM((_NBUF, _CH, EMBED_DIM), jnp.float32),
            pltpu.SemaphoreType.DMA((_NBUF,)),
            pltpu.SemaphoreType.DMA((_NBUF,)),
        ],
        compiler_params=pltpu.CompilerParams(use_tc_tiling_on_sc=False),
    )
    return kern(x3, table)


def kernel(x, embedding_matrix):
    x3 = x.astype(jnp.int32).reshape(_NW, _NCH, _CH)
    out = _sc_gather(x3, embedding_matrix)
    return out.reshape(BATCH, HIST, EMBED_DIM)


# speedup vs baseline: 1.1694x; 1.1694x over previous
import jax
import jax.numpy as jnp
from jax import lax
from jax.experimental import pallas as pl
from jax.experimental.pallas import tpu as pltpu
from jax.experimental.pallas import tpu_sc as plsc

VOCAB = 1000000
EMBED_DIM = 32
BATCH = 4096
HIST = 50

_NC = 2
_NS = 16
_NW = _NC * _NS

_CH = 128
_NCH = HIST
_NBUF = 2


def _gather_body(x_hbm, table_hbm, out_hbm, idx_v, rows_v, t_v, g_sems, o_sems):
    wid = lax.axis_index("s") * _NC + lax.axis_index("c")

    pltpu.sync_copy(x_hbm.at[:, wid], idx_v)

    iotas = [
        lax.broadcasted_iota(jnp.int32, (16,), 0) + g * 16 for g in range(8)
    ]

    def start_gather(j, b):
        pltpu.async_copy(table_hbm.at[idx_v.at[j]], rows_v.at[b], g_sems.at[b])

    def wait_gather(j, b):
        pltpu.make_async_copy(
            table_hbm.at[idx_v.at[j]], rows_v.at[b], g_sems.at[b]
        ).wait()

    def transpose(b):
        for d in range(EMBED_DIM):
            col = jnp.full((16,), d, jnp.int32)
            for g in range(8):
                vec = plsc.load_gather(rows_v.at[b], [iotas[g], col])
                t_v[b, d // 8, d % 8, pl.ds(g * 16, 16)] = vec

    def start_out(j, b):
        pltpu.async_copy(t_v.at[b], out_hbm.at[j, :, wid], o_sems.at[b])

    def wait_out(j, b):
        pltpu.make_async_copy(
            t_v.at[b], out_hbm.at[j, :, wid], o_sems.at[b]
        ).wait()

    for b in range(_NBUF):
        start_gather(b, b)

    def step(g, _):
        for b in range(_NBUF):
            j = g * _NBUF + b
            wait_gather(j, b)

            @pl.when(j >= _NBUF)
            def _():
                wait_out(j - _NBUF, b)

            transpose(b)

            @pl.when(j + _NBUF < _NCH)
            def _():
                start_gather(j + _NBUF, b)

            start_out(j, b)
        return _

    lax.fori_loop(0, _NCH // _NBUF, step, None)

    for b in range(_NBUF):
        wait_out(_NCH - _NBUF + b, b)


def _sc_gather(x3, table):
    kern = pl.kernel(
        _gather_body,
        out_type=jax.ShapeDtypeStruct((HIST, 4, _NW, 8, _CH), jnp.float32),
        mesh=plsc.VectorSubcoreMesh(core_axis_name="c", subcore_axis_name="s"),
        scratch_types=[
            pltpu.VMEM((_NCH, _CH), jnp.int32),
            pltpu.VMEM((_NBUF, _CH, EMBED_DIM), jnp.float32),
            pltpu.VMEM((_NBUF, 4, 8, _CH), jnp.float32),
            pltpu.SemaphoreType.DMA((_NBUF,)),
            pltpu.SemaphoreType.DMA((_NBUF,)),
        ],
        compiler_params=pltpu.CompilerParams(
            use_tc_tiling_on_sc=False, needs_layout_passes=False
        ),
    )
    return kern(x3, table)


def kernel(x, embedding_matrix):
    x3 = x.astype(jnp.int32).T.reshape(HIST, _NW, _CH)
    out5 = _sc_gather(x3, embedding_matrix)
    return out5.transpose(2, 4, 0, 1, 3).reshape(BATCH, HIST, EMBED_DIM)


# speedup vs baseline: 1.3310x; 1.1382x over previous
import jax
import jax.numpy as jnp
from jax import lax
from jax.experimental import pallas as pl
from jax.experimental.pallas import tpu as pltpu
from jax.experimental.pallas import tpu_sc as plsc

VOCAB = 1000000
EMBED_DIM = 32
BATCH = 4096
HIST = 50

_NC = 2
_NS = 16
_NW = _NC * _NS

_CH = 128
_NCH = HIST
_NBUF = 2


def _gather_body(x_hbm, table_hbm, out_hbm, idx_v, rows_v, t_v, g_sems, o_sems):
    wid = lax.axis_index("s") * _NC + lax.axis_index("c")

    pltpu.sync_copy(x_hbm.at[:, wid], idx_v)

    iotas = [
        lax.broadcasted_iota(jnp.int32, (16,), 0) + g * 16 for g in range(8)
    ]
    cols = [jnp.full((16,), d, jnp.int32) for d in range(EMBED_DIM)]

    def start_gather(j, b):
        pltpu.async_copy(table_hbm.at[idx_v.at[j]], rows_v.at[b], g_sems.at[b])

    def wait_gather(j, b):
        pltpu.make_async_copy(
            table_hbm.at[idx_v.at[j]], rows_v.at[b], g_sems.at[b]
        ).wait()

    def transpose(b):
        for g in range(8):
            vecs = [
                plsc.load_gather(rows_v.at[b], [iotas[g], cols[d]])
                for d in range(EMBED_DIM)
            ]
            for d in range(EMBED_DIM):
                t_v[b, d // 8, d % 8, pl.ds(g * 16, 16)] = vecs[d]

    def start_out(j, b):
        pltpu.async_copy(t_v.at[b], out_hbm.at[j, :, wid], o_sems.at[b])

    def wait_out(j, b):
        pltpu.make_async_copy(
            t_v.at[b], out_hbm.at[j, :, wid], o_sems.at[b]
        ).wait()

    for b in range(_NBUF):
        start_gather(b, b)

    def step(g, _):
        for b in range(_NBUF):
            j = g * _NBUF + b
            wait_gather(j, b)

            @pl.when(j >= _NBUF)
            def _():
                wait_out(j - _NBUF, b)

            transpose(b)

            @pl.when(j + _NBUF < _NCH)
            def _():
                start_gather(j + _NBUF, b)

            start_out(j, b)
        return _

    lax.fori_loop(0, _NCH // _NBUF, step, None)

    for b in range(_NBUF):
        wait_out(_NCH - _NBUF + b, b)


def _sc_gather(x3, table):
    kern = pl.kernel(
        _gather_body,
        out_type=jax.ShapeDtypeStruct((HIST, 4, _NW, 8, _CH), jnp.float32),
        mesh=plsc.VectorSubcoreMesh(core_axis_name="c", subcore_axis_name="s"),
        scratch_types=[
            pltpu.VMEM((_NCH, _CH), jnp.int32),
            pltpu.VMEM((_NBUF, _CH, EMBED_DIM), jnp.float32),
            pltpu.VMEM((_NBUF, 4, 8, _CH), jnp.float32),
            pltpu.SemaphoreType.DMA((_NBUF,)),
            pltpu.SemaphoreType.DMA((_NBUF,)),
        ],
        compiler_params=pltpu.CompilerParams(
            use_tc_tiling_on_sc=False, needs_layout_passes=False
        ),
    )
    return kern(x3, table)


def kernel(x, embedding_matrix):
    x3 = x.astype(jnp.int32).T.reshape(HIST, _NW, _CH)
    out5 = _sc_gather(x3, embedding_matrix)
    return out5.transpose(2, 4, 0, 1, 3).reshape(BATCH, HIST, EMBED_DIM)
